# baseline (speedup 1.0000x reference)
import functools
import jax
import jax.numpy as jnp
from jax import lax
from jax.experimental import pallas as pl
from jax.experimental.pallas import tpu as pltpu
from jax.experimental.pallas import tpu_sc as plsc

_NC = 2
_NS = 16
_NW = _NC * _NS
_L = 16

_C = 16


def _tc_add_kernel(xa_ref, xb_ref, pa_ref, pb_ref, o_ref):
    h = xa_ref.shape[1]
    o_ref[0, :h, :] = xa_ref[0] + pa_ref[...]
    o_ref[0, h:, :] = xb_ref[0] + pb_ref[...]


def _tc_part(x, pe, nb):
    b, s, d = x.shape
    h = s // 2
    return pl.pallas_call(
        _tc_add_kernel,
        grid=(nb,),
        in_specs=[
            pl.BlockSpec((1, h, d), lambda j: (j, 0, 0)),
            pl.BlockSpec((1, h, d), lambda j: (j, 1, 0)),
            pl.BlockSpec((h, d), lambda j: (0, 0)),
            pl.BlockSpec((h, d), lambda j: (1, 0)),
        ],
        out_specs=pl.BlockSpec((1, s, d), lambda j: (j, 0, 0)),
        out_shape=jax.ShapeDtypeStruct((nb, s, d), x.dtype),
        compiler_params=pltpu.CompilerParams(vmem_limit_bytes=100 * 1024 * 1024),
    )(x, x, pe, pe)


def _sc_body(x_hbm, pe_hbm, o_hbm,
             xb0, xb1, xb2, pb0, pb1, pb2,
             sx0, sx1, sx2, sp0, sp1, sp2, so0, so1, so2):
    b, s, d = x_hbm.shape
    bi = b - 1
    rows_per_w = s // _NW
    n_u = rows_per_w // _C
    wid = lax.axis_index("s") * _NC + lax.axis_index("c")
    s0 = wid * rows_per_w

    xb, pb = [xb0, xb1, xb2], [pb0, pb1, pb2]
    sx, sp, so = [sx0, sx1, sx2], [sp0, sp1, sp2], [so0, so1, so2]

    inx_h, inp_h, out_h = {}, {}, {}
    for t in range(n_u + 1):
        if t < n_u:
            if t >= 3:
                out_h[t - 3].wait()
            row = pl.ds(s0 + t * _C, _C)
            inx_h[t] = pltpu.async_copy(x_hbm.at[bi, row], xb[t % 3], sx[t % 3])
            inp_h[t] = pltpu.async_copy(pe_hbm.at[row], pb[t % 3], sp[t % 3])
        if 0 <= t - 1 < n_u:
            u = t - 1
            inx_h[u].wait()
            inp_h[u].wait()
            buf, pe_buf = xb[u % 3], pb[u % 3]

            @plsc.parallel_loop(0, _C, 1, unroll=2)
            def add_row(r, buf=buf, pe_buf=pe_buf):
                for j in range(d // _L):
                    sl = pl.ds(j * _L, _L)
                    buf[r, sl] = buf[r, sl] + pe_buf[r, sl]

            out_h[u] = pltpu.async_copy(
                buf, o_hbm.at[0, pl.ds(s0 + u * _C, _C)], so[u % 3])

    out_h[n_u - 3].wait()
    out_h[n_u - 2].wait()
    out_h[n_u - 1].wait()


def _sc_part(x, pe):
    b, s, d = x.shape
    mesh = plsc.VectorSubcoreMesh(core_axis_name="c", subcore_axis_name="s")
    k = functools.partial(
        pl.kernel,
        mesh=mesh,
        out_type=jax.ShapeDtypeStruct((1, s, d), x.dtype),
        scratch_types=(
            [pltpu.VMEM((_C, d), jnp.float32)] * 6
            + [pltpu.SemaphoreType.DMA] * 9
        ),
    )(_sc_body)
    return k(x, pe)


def kernel(x, pos_emb):
    b, s, d = x.shape
    pe = pos_emb[:s]
    out_tc = _tc_part(x, pe, b - 1)
    out_sc = _sc_part(x, pe)
    return jnp.concatenate([out_tc, out_sc], axis=0)
